# Initial kernel scaffold; baseline (speedup 1.0000x reference)
import functools

import jax
import jax.numpy as jnp
from jax import lax
from jax.experimental import pallas as pl
from jax.experimental.pallas import tpu as pltpu
from jax.experimental.pallas import tpu_sc as plsc

N = 10000
E = 320000
D = 128
R = 7
T = 4
KEYS = R * N
C = 128
CHUNKS = E // C
NC = 2
NS = 16
NW = NC * NS
ITERS = (CHUNKS + NW - 1) // NW
ROWS_PER_TILE = KEYS // NS
NODES_PER_TILE = N // NS

_mesh = plsc.VectorSubcoreMesh(core_axis_name="c", subcore_axis_name="s")
_f32 = jnp.float32
_i32 = jnp.int32


def _tc_pre_body(xs_ref, xt_ref, wr_ref, wroot_ref, broot_ref, tnt_ref,
                 z_ref, root_ref):
    xs = xs_ref[...]
    dn = (((1,), (1,)), ((), ()))
    for r in range(R):
        z_ref[r] = lax.dot_general(xs, wr_ref[r], dn,
                                   precision=lax.Precision.HIGHEST,
                                   preferred_element_type=_f32)
    xt = xt_ref[...]
    tt = tnt_ref[...]
    acc = jnp.zeros_like(xt)
    for i in range(T):
        v = lax.dot_general(xt, wroot_ref[i], dn,
                            precision=lax.Precision.HIGHEST,
                            preferred_element_type=_f32) + broot_ref[i][None, :]
        acc = acc + jnp.where(tt == i, 1.0, 0.0).astype(_f32) * v
    root_ref[...] = acc


def _tc_precompute(x_src, x_target, W_rel, W_root, b_root, tnt2d):
    nb = 10
    bn = N // nb
    return pl.pallas_call(
        _tc_pre_body,
        grid=(nb,),
        in_specs=[
            pl.BlockSpec((bn, D), lambda i: (i, 0)),
            pl.BlockSpec((bn, D), lambda i: (i, 0)),
            pl.BlockSpec((R, D, D), lambda i: (0, 0, 0)),
            pl.BlockSpec((T, D, D), lambda i: (0, 0, 0)),
            pl.BlockSpec((T, D), lambda i: (0, 0)),
            pl.BlockSpec((bn, 1), lambda i: (i, 0)),
        ],
        out_specs=[
            pl.BlockSpec((R, bn, D), lambda i: (0, i, 0)),
            pl.BlockSpec((bn, D), lambda i: (i, 0)),
        ],
        out_shape=[
            jax.ShapeDtypeStruct((R, N, D), _f32),
            jax.ShapeDtypeStruct((N, D), _f32),
        ],
    )(x_src, x_target, W_rel, W_root, b_root, tnt2d)


def _sc_count_body(dst_hbm, typ_hbm, out_hbm, dstv, typv, keyv, onesv, zerov,
                   cnt_sh, sem):
    core = lax.axis_index("c")
    sub = lax.axis_index("s")
    wid = sub * NC + core
    lane = lax.iota(_i32, 16)
    one_row = jnp.where(lane == 0, 1.0, 0.0).astype(_f32)
    zero_row = jnp.zeros((16,), _f32)

    @pl.loop(0, C)
    def _(j):
        onesv[j] = one_row

    @pl.loop(0, ROWS_PER_TILE)
    def _(j):
        zerov[j] = zero_row

    pltpu.sync_copy(zerov, cnt_sh.at[pl.ds(sub * ROWS_PER_TILE, ROWS_PER_TILE)])
    plsc.subcore_barrier()

    @pl.loop(0, ITERS)
    def _(it):
        cid = wid + it * NW

        @pl.when(cid < CHUNKS)
        def _():
            base = cid * C
            pltpu.sync_copy(dst_hbm.at[pl.ds(base, C)], dstv)
            pltpu.sync_copy(typ_hbm.at[pl.ds(base, C)], typv)

            @pl.loop(0, C, step=16)
            def _(g):
                keyv[pl.ds(g, 16)] = typv[pl.ds(g, 16)] * N + dstv[pl.ds(g, 16)]

            pltpu.sync_copy(onesv, cnt_sh.at[keyv], add=True)

    plsc.subcore_barrier()
    pltpu.sync_copy(cnt_sh.at[pl.ds(sub * ROWS_PER_TILE, ROWS_PER_TILE)],
                    out_hbm.at[core, pl.ds(sub * ROWS_PER_TILE, ROWS_PER_TILE)])


_sc_count = pl.kernel(
    _sc_count_body,
    out_type=jax.ShapeDtypeStruct((NC, KEYS, 16), _f32),
    mesh=_mesh,
    scratch_types=[
        pltpu.VMEM((C,), _i32),
        pltpu.VMEM((C,), _i32),
        pltpu.VMEM((C,), _i32),
        pltpu.VMEM((C, 16), _f32),
        pltpu.VMEM((ROWS_PER_TILE, 16), _f32),
        pltpu.VMEM_SHARED((KEYS, 16), _f32),
        pltpu.SemaphoreType.DMA,
    ],
)


def _tc_inv_body(cnt_ref, inv_ref):
    a = cnt_ref[...]
    c = jnp.sum(a[0] + a[1], axis=1)
    inv_ref[...] = (1.0 / jnp.maximum(c, 1.0))[None, :]


def _tc_inv(cnt):
    return pl.pallas_call(
        _tc_inv_body,
        out_shape=jax.ShapeDtypeStruct((1, KEYS), _f32),
    )(cnt)


def _sc_main_body(src_hbm, dst_hbm, typ_hbm, z_hbm, inv_hbm, out_hbm,
                  invv, srcv, dstv, typv, gkeyv, scalev, rows, zbuf,
                  acc_sh, sem):
    core = lax.axis_index("c")
    sub = lax.axis_index("s")
    wid = sub * NC + core
    zero_row = jnp.zeros((16,), _f32)

    @pl.loop(0, 125)
    def _(j):
        for k in range(8):
            zbuf[j, pl.ds(k * 16, 16)] = zero_row

    for k in range(5):
        pltpu.sync_copy(zbuf, acc_sh.at[pl.ds(sub * NODES_PER_TILE + k * 125, 125)])
    pltpu.sync_copy(inv_hbm, invv)
    plsc.subcore_barrier()

    @pl.loop(0, ITERS)
    def _(it):
        cid = wid + it * NW

        @pl.when(cid < CHUNKS)
        def _():
            base = cid * C
            pltpu.sync_copy(src_hbm.at[pl.ds(base, C)], srcv)
            pltpu.sync_copy(dst_hbm.at[pl.ds(base, C)], dstv)
            pltpu.sync_copy(typ_hbm.at[pl.ds(base, C)], typv)

            @pl.loop(0, C, step=16)
            def _(g):
                t16 = typv[pl.ds(g, 16)]
                gkeyv[pl.ds(g, 16)] = t16 * N + srcv[pl.ds(g, 16)]
                ck = t16 * N + dstv[pl.ds(g, 16)]
                scalev[pl.ds(g, 16)] = plsc.load_gather(invv, [ck])

            pltpu.async_copy(z_hbm.at[gkeyv], rows, sem).wait()

            @pl.loop(0, C)
            def _(j):
                jj = lax.broadcast(j, (16,))
                s16 = plsc.load_gather(scalev, [jj])
                for k in range(8):
                    rows[j, pl.ds(k * 16, 16)] = rows[j, pl.ds(k * 16, 16)] * s16

            pltpu.sync_copy(rows, acc_sh.at[dstv], add=True)

    plsc.subcore_barrier()
    pltpu.sync_copy(acc_sh.at[pl.ds(sub * NODES_PER_TILE, NODES_PER_TILE)],
                    out_hbm.at[core, pl.ds(sub * NODES_PER_TILE, NODES_PER_TILE)])


_sc_main = pl.kernel(
    _sc_main_body,
    out_type=jax.ShapeDtypeStruct((NC, N, D), _f32),
    mesh=_mesh,
    scratch_types=[
        pltpu.VMEM((KEYS,), _f32),
        pltpu.VMEM((C,), _i32),
        pltpu.VMEM((C,), _i32),
        pltpu.VMEM((C,), _i32),
        pltpu.VMEM((C,), _i32),
        pltpu.VMEM((C,), _f32),
        pltpu.VMEM((C, D), _f32),
        pltpu.VMEM((125, D), _f32),
        pltpu.VMEM_SHARED((N, D), _f32),
        pltpu.SemaphoreType.DMA,
    ],
)


def _tc_final_body(part_ref, root_ref, out_ref):
    p = part_ref[...]
    out_ref[...] = p[0] + p[1] + root_ref[...]


def _tc_final(parts, root):
    nb = 10
    bn = N // nb
    return pl.pallas_call(
        _tc_final_body,
        grid=(nb,),
        in_specs=[
            pl.BlockSpec((NC, bn, D), lambda i: (0, i, 0)),
            pl.BlockSpec((bn, D), lambda i: (i, 0)),
        ],
        out_specs=pl.BlockSpec((bn, D), lambda i: (i, 0)),
        out_shape=jax.ShapeDtypeStruct((N, D), _f32),
    )(parts, root)


def kernel(x_src, x_target, edge_index, edge_type, target_node_type,
           src_node_type, W_rel, W_root, b_root):
    src = edge_index[0].astype(_i32)
    dst = edge_index[1].astype(_i32)
    typ = edge_type.astype(_i32)
    tnt2d = target_node_type.astype(_i32).reshape(N, 1)

    z, root = _tc_precompute(x_src.astype(_f32), x_target.astype(_f32),
                             W_rel.astype(_f32), W_root.astype(_f32),
                             b_root.astype(_f32), tnt2d)
    z = z.reshape(KEYS, D)

    cnt = _sc_count(dst, typ)
    inv = _tc_inv(cnt).reshape(KEYS)

    parts = _sc_main(src, dst, typ, z, inv)
    return _tc_final(parts, root)


# speedup vs baseline: 13.7014x; 13.7014x over previous
import dataclasses
import functools

import jax
import jax.numpy as jnp
from jax import lax
from jax.experimental import pallas as pl
from jax.experimental.pallas import tpu as pltpu
from jax.experimental.pallas import tpu_sc as plsc

N = 10000
E = 320000
D = 128
R = 7
T = 4
KEYS = R * N
KEYS_PAD = 70144
C = 128
CHUNKS = E // C
NC = 2
NS = 16
NW = NC * NS
ITERS = (CHUNKS + NW - 1) // NW
ELEMS_PER_TILE = KEYS_PAD // NS
NPT = 624

_mesh = plsc.VectorSubcoreMesh(core_axis_name="c", subcore_axis_name="s")
_f32 = jnp.float32
_i32 = jnp.int32

_sc_params = pltpu.CompilerParams()
if "needs_layout_passes" in pltpu.CompilerParams.__dataclass_fields__:
    _sc_params = dataclasses.replace(_sc_params, needs_layout_passes=False)


def _tc_pre_body(xs_ref, xt_ref, wr_ref, wroot_ref, broot_ref, tnt_ref,
                 z_ref, root_ref):
    xs = xs_ref[...]
    dn = (((1,), (1,)), ((), ()))
    for r in range(R):
        z_ref[r] = lax.dot_general(xs, wr_ref[r], dn,
                                   precision=lax.Precision.HIGHEST,
                                   preferred_element_type=_f32)
    xt = xt_ref[...]
    tt = tnt_ref[...]
    acc = jnp.zeros_like(xt)
    for i in range(T):
        v = lax.dot_general(xt, wroot_ref[i], dn,
                            precision=lax.Precision.HIGHEST,
                            preferred_element_type=_f32) + broot_ref[i][None, :]
        acc = acc + jnp.where(tt == i, 1.0, 0.0).astype(_f32) * v
    root_ref[...] = acc


def _tc_precompute(x_src, x_target, W_rel, W_root, b_root, tnt2d):
    nb = 10
    bn = N // nb
    return pl.pallas_call(
        _tc_pre_body,
        grid=(nb,),
        in_specs=[
            pl.BlockSpec((bn, D), lambda i: (i, 0)),
            pl.BlockSpec((bn, D), lambda i: (i, 0)),
            pl.BlockSpec((R, D, D), lambda i: (0, 0, 0)),
            pl.BlockSpec((T, D, D), lambda i: (0, 0, 0)),
            pl.BlockSpec((T, D), lambda i: (0, 0)),
            pl.BlockSpec((bn, 1), lambda i: (i, 0)),
        ],
        out_specs=[
            pl.BlockSpec((R, bn, D), lambda i: (0, i, 0)),
            pl.BlockSpec((bn, D), lambda i: (i, 0)),
        ],
        out_shape=[
            jax.ShapeDtypeStruct((R, N, D), _f32),
            jax.ShapeDtypeStruct((N, D), _f32),
        ],
    )(x_src, x_target, W_rel, W_root, b_root, tnt2d)


def _sc_count_body(dst_hbm, typ_hbm, out_hbm, dstv, typv, keyv, onesv, zerov,
                   cnt_sh, sem):
    core = lax.axis_index("c")
    sub = lax.axis_index("s")
    wid = sub * NC + core
    ones16 = jnp.full((16,), 1.0, _f32)
    zero16 = jnp.zeros((16,), _f32)

    @pl.loop(0, C, step=16)
    def _(g):
        onesv[pl.ds(g, 16)] = ones16

    @pl.loop(0, ELEMS_PER_TILE, step=16)
    def _(g):
        zerov[pl.ds(g, 16)] = zero16

    pltpu.sync_copy(zerov, cnt_sh.at[pl.ds(sub * ELEMS_PER_TILE, ELEMS_PER_TILE)])
    plsc.subcore_barrier()

    @pl.loop(0, ITERS)
    def _(it):
        cid = wid + it * NW

        @pl.when(cid < CHUNKS)
        def _():
            base = cid * C
            pltpu.sync_copy(dst_hbm.at[pl.ds(base, C)], dstv)
            pltpu.sync_copy(typ_hbm.at[pl.ds(base, C)], typv)

            @pl.loop(0, C, step=16)
            def _(g):
                keyv[pl.ds(g, 16)] = typv[pl.ds(g, 16)] * N + dstv[pl.ds(g, 16)]

            pltpu.sync_copy(onesv, cnt_sh.at[keyv], add=True)

    plsc.subcore_barrier()
    pltpu.sync_copy(cnt_sh.at[pl.ds(sub * ELEMS_PER_TILE, ELEMS_PER_TILE)], zerov)
    pltpu.sync_copy(zerov,
                    out_hbm.at[pl.ds(core * KEYS_PAD + sub * ELEMS_PER_TILE,
                                     ELEMS_PER_TILE)])


_sc_count = pl.kernel(
    _sc_count_body,
    out_type=jax.ShapeDtypeStruct((NC * KEYS_PAD,), _f32),
    mesh=_mesh,
    scratch_types=[
        pltpu.VMEM((C,), _i32),
        pltpu.VMEM((C,), _i32),
        pltpu.VMEM((C,), _i32),
        pltpu.VMEM((C,), _f32),
        pltpu.VMEM((ELEMS_PER_TILE,), _f32),
        pltpu.VMEM_SHARED((KEYS_PAD,), _f32),
        pltpu.SemaphoreType.DMA,
    ],
)


def _tc_inv_body(cnt_ref, inv_ref):
    a = cnt_ref[...]
    c = a[0] + a[1]
    inv_ref[...] = (1.0 / jnp.maximum(c, 1.0))[None, :]


def _tc_inv(cnt):
    return pl.pallas_call(
        _tc_inv_body,
        out_shape=jax.ShapeDtypeStruct((1, KEYS_PAD), _f32),
    )(cnt)


def _sc_main_body(src_hbm, dst_hbm, typ_hbm, z_hbm, inv_hbm, out_hbm,
                  srcv, dstv, typv, gkeyv, ckeyv, scalev, rows,
                  acc_sh, sem):
    core = lax.axis_index("c")
    sub = lax.axis_index("s")
    wid = sub * NC + core
    zero_row = jnp.zeros((16,), _f32)

    @pl.loop(0, C)
    def _(j):
        for k in range(8):
            rows[j, pl.ds(k * 16, 16)] = zero_row

    for k in range(4):
        pltpu.sync_copy(rows, acc_sh.at[pl.ds(sub * NPT + k * 128, 128)])
    pltpu.sync_copy(rows.at[pl.ds(0, 112)],
                    acc_sh.at[pl.ds(sub * NPT + 512, 112)])

    @pl.when(sub == NS - 1)
    def _():
        pltpu.sync_copy(rows.at[pl.ds(0, 16)], acc_sh.at[pl.ds(NS * NPT, 16)])

    plsc.subcore_barrier()

    @pl.loop(0, ITERS)
    def _(it):
        cid = wid + it * NW

        @pl.when(cid < CHUNKS)
        def _():
            base = cid * C
            pltpu.sync_copy(src_hbm.at[pl.ds(base, C)], srcv)
            pltpu.sync_copy(dst_hbm.at[pl.ds(base, C)], dstv)
            pltpu.sync_copy(typ_hbm.at[pl.ds(base, C)], typv)

            @pl.loop(0, C, step=16)
            def _(g):
                t16 = typv[pl.ds(g, 16)]
                gkeyv[pl.ds(g, 16)] = t16 * N + srcv[pl.ds(g, 16)]
                ckeyv[pl.ds(g, 16)] = t16 * N + dstv[pl.ds(g, 16)]

            pltpu.sync_copy(inv_hbm.at[ckeyv], scalev)
            pltpu.async_copy(z_hbm.at[gkeyv], rows, sem).wait()

            @pl.loop(0, C)
            def _(j):
                jj = lax.broadcast(j, (16,))
                s16 = plsc.load_gather(scalev, [jj])
                for k in range(8):
                    rows[j, pl.ds(k * 16, 16)] = rows[j, pl.ds(k * 16, 16)] * s16

            pltpu.sync_copy(rows, acc_sh.at[dstv], add=True)

    plsc.subcore_barrier()
    for k in range(4):
        pltpu.sync_copy(acc_sh.at[pl.ds(sub * NPT + k * 128, 128)], rows)
        pltpu.sync_copy(rows, out_hbm.at[core, pl.ds(sub * NPT + k * 128, 128)])
    pltpu.sync_copy(acc_sh.at[pl.ds(sub * NPT + 512, 112)],
                    rows.at[pl.ds(0, 112)])
    pltpu.sync_copy(rows.at[pl.ds(0, 112)],
                    out_hbm.at[core, pl.ds(sub * NPT + 512, 112)])

    @pl.when(sub == NS - 1)
    def _():
        pltpu.sync_copy(acc_sh.at[pl.ds(NS * NPT, 16)], rows.at[pl.ds(0, 16)])
        pltpu.sync_copy(rows.at[pl.ds(0, 16)],
                        out_hbm.at[core, pl.ds(NS * NPT, 16)])


_sc_main = pl.kernel(
    _sc_main_body,
    out_type=jax.ShapeDtypeStruct((NC, N, D), _f32),
    mesh=_mesh,
    scratch_types=[
        pltpu.VMEM((C,), _i32),
        pltpu.VMEM((C,), _i32),
        pltpu.VMEM((C,), _i32),
        pltpu.VMEM((C,), _i32),
        pltpu.VMEM((C,), _i32),
        pltpu.VMEM((C,), _f32),
        pltpu.VMEM((C, D), _f32),
        pltpu.VMEM_SHARED((N, D), _f32),
        pltpu.SemaphoreType.DMA,
    ],
    compiler_params=_sc_params,
)


def _tc_final_body(part_ref, root_ref, out_ref):
    p = part_ref[...]
    out_ref[...] = p[0] + p[1] + root_ref[...]


def _tc_final(parts, root):
    nb = 10
    bn = N // nb
    return pl.pallas_call(
        _tc_final_body,
        grid=(nb,),
        in_specs=[
            pl.BlockSpec((NC, bn, D), lambda i: (0, i, 0)),
            pl.BlockSpec((bn, D), lambda i: (i, 0)),
        ],
        out_specs=pl.BlockSpec((bn, D), lambda i: (i, 0)),
        out_shape=jax.ShapeDtypeStruct((N, D), _f32),
    )(parts, root)


def kernel(x_src, x_target, edge_index, edge_type, target_node_type,
           src_node_type, W_rel, W_root, b_root):
    src = edge_index[0].astype(_i32)
    dst = edge_index[1].astype(_i32)
    typ = edge_type.astype(_i32)
    tnt2d = target_node_type.astype(_i32).reshape(N, 1)

    z, root = _tc_precompute(x_src.astype(_f32), x_target.astype(_f32),
                             W_rel.astype(_f32), W_root.astype(_f32),
                             b_root.astype(_f32), tnt2d)
    z = z.reshape(KEYS, D)

    cnt = _sc_count(dst, typ).reshape(NC, KEYS_PAD)
    inv = _tc_inv(cnt).reshape(KEYS_PAD)

    parts = _sc_main(src, dst, typ, z, inv)
    return _tc_final(parts, root)


# speedup vs baseline: 25.1514x; 1.8357x over previous
import dataclasses
import functools

import jax
import jax.numpy as jnp
from jax import lax
from jax.experimental import pallas as pl
from jax.experimental.pallas import tpu as pltpu
from jax.experimental.pallas import tpu_sc as plsc

N = 10000
E = 320000
D = 128
R = 7
T = 4
KEYS = R * N
KEYS_PAD = 70144
C = 128
CHUNKS = E // C
NC = 2
NS = 16
NW = NC * NS
ITERS = (CHUNKS + NW - 1) // NW
PAIRS = (ITERS + 1) // 2
ELEMS_PER_TILE = KEYS_PAD // NS
NPT = 624

_mesh = plsc.VectorSubcoreMesh(core_axis_name="c", subcore_axis_name="s")
_f32 = jnp.float32
_i32 = jnp.int32

_sc_params = pltpu.CompilerParams()
if "needs_layout_passes" in pltpu.CompilerParams.__dataclass_fields__:
    _sc_params = dataclasses.replace(_sc_params, needs_layout_passes=False)


def _tc_pre_body(xs_ref, xt_ref, wr_ref, wroot_ref, broot_ref, tnt_ref,
                 z_ref, root_ref):
    xs = xs_ref[...]
    dn = (((1,), (1,)), ((), ()))
    for r in range(R):
        z_ref[r] = lax.dot_general(xs, wr_ref[r], dn,
                                   precision=lax.Precision.HIGHEST,
                                   preferred_element_type=_f32)
    xt = xt_ref[...]
    tt = tnt_ref[...]
    acc = jnp.zeros_like(xt)
    for i in range(T):
        v = lax.dot_general(xt, wroot_ref[i], dn,
                            precision=lax.Precision.HIGHEST,
                            preferred_element_type=_f32) + broot_ref[i][None, :]
        acc = acc + jnp.where(tt == i, 1.0, 0.0).astype(_f32) * v
    root_ref[...] = acc


def _tc_precompute(x_src, x_target, W_rel, W_root, b_root, tnt2d):
    nb = 10
    bn = N // nb
    return pl.pallas_call(
        _tc_pre_body,
        grid=(nb,),
        in_specs=[
            pl.BlockSpec((bn, D), lambda i: (i, 0)),
            pl.BlockSpec((bn, D), lambda i: (i, 0)),
            pl.BlockSpec((R, D, D), lambda i: (0, 0, 0)),
            pl.BlockSpec((T, D, D), lambda i: (0, 0, 0)),
            pl.BlockSpec((T, D), lambda i: (0, 0)),
            pl.BlockSpec((bn, 1), lambda i: (i, 0)),
        ],
        out_specs=[
            pl.BlockSpec((R, bn, D), lambda i: (0, i, 0)),
            pl.BlockSpec((bn, D), lambda i: (i, 0)),
        ],
        out_shape=[
            jax.ShapeDtypeStruct((R, N, D), _f32),
            jax.ShapeDtypeStruct((N, D), _f32),
        ],
    )(x_src, x_target, W_rel, W_root, b_root, tnt2d)


def _tc_keys_body(src_ref, dst_ref, typ_ref, gk_ref, ck_ref):
    t = typ_ref[...]
    gk_ref[...] = t * N + src_ref[...]
    ck_ref[...] = t * N + dst_ref[...]


def _tc_keys(src, dst, typ):
    return pl.pallas_call(
        _tc_keys_body,
        out_shape=[jax.ShapeDtypeStruct((E,), _i32)] * 2,
    )(src, dst, typ)


def _sc_count_body(ckey_hbm, out_hbm, kv0, kv1, onesv, zerov, cnt_sh,
                   si0, si1, so0, so1):
    KV = (kv0, kv1)
    SI = (si0, si1)
    SO = (so0, so1)
    core = lax.axis_index("c")
    sub = lax.axis_index("s")
    wid = sub * NC + core
    ones16 = jnp.full((16,), 1.0, _f32)
    zero16 = jnp.zeros((16,), _f32)

    @pl.loop(0, C, step=16)
    def _(g):
        onesv[pl.ds(g, 16)] = ones16

    @pl.loop(0, ELEMS_PER_TILE, step=16)
    def _(g):
        zerov[pl.ds(g, 16)] = zero16

    pltpu.sync_copy(zerov, cnt_sh.at[pl.ds(sub * ELEMS_PER_TILE, ELEMS_PER_TILE)])
    plsc.subcore_barrier()

    for i0 in (0, 1):
        pltpu.async_copy(ckey_hbm.at[pl.ds((wid + i0 * NW) * C, C)],
                         KV[i0], SI[i0])

    @pl.loop(0, PAIRS)
    def _(p):
        for off in (0, 1):
            b = off
            i = 2 * p + off
            cid = wid + i * NW
            cid2 = cid + 2 * NW

            @pl.when(cid < CHUNKS)
            def _():
                pltpu.make_async_copy(ckey_hbm.at[pl.ds(cid * C, C)],
                                      KV[b], SI[b]).wait()
                pltpu.async_copy(onesv, cnt_sh.at[KV[b]], SO[b], add=True)

            @pl.when(cid2 < CHUNKS)
            def _():
                pltpu.make_async_copy(onesv, cnt_sh.at[KV[b]], SO[b]).wait()
                pltpu.async_copy(ckey_hbm.at[pl.ds(cid2 * C, C)], KV[b], SI[b])

    pltpu.make_async_copy(onesv, cnt_sh.at[KV[0]], SO[0]).wait()
    pltpu.make_async_copy(onesv, cnt_sh.at[KV[1]], SO[1]).wait()

    plsc.subcore_barrier()
    pltpu.sync_copy(cnt_sh.at[pl.ds(sub * ELEMS_PER_TILE, ELEMS_PER_TILE)], zerov)
    pltpu.sync_copy(zerov,
                    out_hbm.at[pl.ds(core * KEYS_PAD + sub * ELEMS_PER_TILE,
                                     ELEMS_PER_TILE)])


_sc_count = pl.kernel(
    _sc_count_body,
    out_type=jax.ShapeDtypeStruct((NC * KEYS_PAD,), _f32),
    mesh=_mesh,
    scratch_types=[
        pltpu.VMEM((C,), _i32),
        pltpu.VMEM((C,), _i32),
        pltpu.VMEM((C,), _f32),
        pltpu.VMEM((ELEMS_PER_TILE,), _f32),
        pltpu.VMEM_SHARED((KEYS_PAD,), _f32),
        pltpu.SemaphoreType.DMA,
        pltpu.SemaphoreType.DMA,
        pltpu.SemaphoreType.DMA,
        pltpu.SemaphoreType.DMA,
    ],
    compiler_params=_sc_params,
)


def _tc_inv_body(cnt_ref, inv_ref):
    a = cnt_ref[...]
    c = a[0] + a[1]
    inv_ref[...] = (1.0 / jnp.maximum(c, 1.0))[None, :]


def _tc_inv(cnt):
    return pl.pallas_call(
        _tc_inv_body,
        out_shape=jax.ShapeDtypeStruct((1, KEYS_PAD), _f32),
    )(cnt)


def _sc_main_body(gkey_hbm, ckey_hbm, dst_hbm, z_hbm, inv_hbm, out_hbm,
                  gk0, gk1, ck0, ck1, dv0, dv1, sd0, sd1, sc0, sc1, r0, r1,
                  si0, si1, ss0, ss1, sz0, sz1, so0, so1, acc_sh):
    GK = (gk0, gk1)
    CK = (ck0, ck1)
    DV = (dv0, dv1)
    SD = (sd0, sd1)
    SCV = (sc0, sc1)
    RW = (r0, r1)
    SI = (si0, si1)
    SS = (ss0, ss1)
    SZ = (sz0, sz1)
    SO = (so0, so1)
    core = lax.axis_index("c")
    sub = lax.axis_index("s")
    wid = sub * NC + core
    zero_row = jnp.zeros((16,), _f32)

    @pl.loop(0, C)
    def _(j):
        for k in range(8):
            r0[j, pl.ds(k * 16, 16)] = zero_row

    for k in range(4):
        pltpu.sync_copy(r0, acc_sh.at[pl.ds(sub * NPT + k * 128, 128)])
    pltpu.sync_copy(r0.at[pl.ds(0, 112)],
                    acc_sh.at[pl.ds(sub * NPT + 512, 112)])

    @pl.when(sub == NS - 1)
    def _():
        pltpu.sync_copy(r0.at[pl.ds(0, 16)], acc_sh.at[pl.ds(NS * NPT, 16)])

    plsc.subcore_barrier()

    def idx_issue(i, b):
        cid = wid + i * NW

        @pl.when(cid < CHUNKS)
        def _():
            base = cid * C
            pltpu.async_copy(gkey_hbm.at[pl.ds(base, C)], GK[b], SI[b])
            pltpu.async_copy(ckey_hbm.at[pl.ds(base, C)], CK[b], SI[b])
            pltpu.async_copy(dst_hbm.at[pl.ds(base, C)], DV[b], SI[b])

    def gathers_issue(i, b):
        cid = wid + i * NW

        @pl.when(cid < CHUNKS)
        def _():
            base = cid * C
            pltpu.make_async_copy(gkey_hbm.at[pl.ds(base, C)], GK[b], SI[b]).wait()
            pltpu.make_async_copy(ckey_hbm.at[pl.ds(base, C)], CK[b], SI[b]).wait()
            pltpu.make_async_copy(dst_hbm.at[pl.ds(base, C)], DV[b], SI[b]).wait()

            @pl.when(i >= 2)
            def _():
                pltpu.make_async_copy(RW[b], acc_sh.at[SD[b]], SO[b]).wait()

            pltpu.async_copy(inv_hbm.at[CK[b]], SCV[b], SS[b])
            pltpu.async_copy(z_hbm.at[GK[b]], RW[b], SZ[b])

    def process(i, b):
        cid = wid + i * NW

        @pl.when(cid < CHUNKS)
        def _():
            pltpu.make_async_copy(inv_hbm.at[CK[b]], SCV[b], SS[b]).wait()
            pltpu.make_async_copy(z_hbm.at[GK[b]], RW[b], SZ[b]).wait()

            @pl.loop(0, C, step=16)
            def _(g):
                SD[b][pl.ds(g, 16)] = DV[b][pl.ds(g, 16)]

        idx_issue(i + 2, b)

        @pl.when(cid < CHUNKS)
        def _():
            @pl.loop(0, C)
            def _(j):
                jj = lax.broadcast(j, (16,))
                s16 = plsc.load_gather(SCV[b], [jj])
                for k in range(8):
                    RW[b][j, pl.ds(k * 16, 16)] = RW[b][j, pl.ds(k * 16, 16)] * s16

            pltpu.async_copy(RW[b], acc_sh.at[SD[b]], SO[b], add=True)

    idx_issue(0, 0)
    idx_issue(1, 1)
    gathers_issue(0, 0)

    @pl.loop(0, PAIRS)
    def _(p):
        for off in (0, 1):
            i = 2 * p + off
            gathers_issue(i + 1, (off + 1) % 2)
            process(i, off)

    pltpu.make_async_copy(RW[0], acc_sh.at[SD[0]], SO[0]).wait()
    pltpu.make_async_copy(RW[1], acc_sh.at[SD[1]], SO[1]).wait()

    plsc.subcore_barrier()
    for k in range(4):
        pltpu.sync_copy(acc_sh.at[pl.ds(sub * NPT + k * 128, 128)], r0)
        pltpu.sync_copy(r0, out_hbm.at[core, pl.ds(sub * NPT + k * 128, 128)])
    pltpu.sync_copy(acc_sh.at[pl.ds(sub * NPT + 512, 112)], r0.at[pl.ds(0, 112)])
    pltpu.sync_copy(r0.at[pl.ds(0, 112)],
                    out_hbm.at[core, pl.ds(sub * NPT + 512, 112)])

    @pl.when(sub == NS - 1)
    def _():
        pltpu.sync_copy(acc_sh.at[pl.ds(NS * NPT, 16)], r0.at[pl.ds(0, 16)])
        pltpu.sync_copy(r0.at[pl.ds(0, 16)],
                        out_hbm.at[core, pl.ds(NS * NPT, 16)])


_sc_main = pl.kernel(
    _sc_main_body,
    out_type=jax.ShapeDtypeStruct((NC, N, D), _f32),
    mesh=_mesh,
    scratch_types=[
        pltpu.VMEM((C,), _i32),
        pltpu.VMEM((C,), _i32),
        pltpu.VMEM((C,), _i32),
        pltpu.VMEM((C,), _i32),
        pltpu.VMEM((C,), _i32),
        pltpu.VMEM((C,), _i32),
        pltpu.VMEM((C,), _i32),
        pltpu.VMEM((C,), _i32),
        pltpu.VMEM((C,), _f32),
        pltpu.VMEM((C,), _f32),
        pltpu.VMEM((C, D), _f32),
        pltpu.VMEM((C, D), _f32),
        pltpu.SemaphoreType.DMA,
        pltpu.SemaphoreType.DMA,
        pltpu.SemaphoreType.DMA,
        pltpu.SemaphoreType.DMA,
        pltpu.SemaphoreType.DMA,
        pltpu.SemaphoreType.DMA,
        pltpu.SemaphoreType.DMA,
        pltpu.SemaphoreType.DMA,
        pltpu.VMEM_SHARED((N, D), _f32),
    ],
    compiler_params=_sc_params,
)


def _tc_final_body(part_ref, root_ref, out_ref):
    p = part_ref[...]
    out_ref[...] = p[0] + p[1] + root_ref[...]


def _tc_final(parts, root):
    nb = 10
    bn = N // nb
    return pl.pallas_call(
        _tc_final_body,
        grid=(nb,),
        in_specs=[
            pl.BlockSpec((NC, bn, D), lambda i: (0, i, 0)),
            pl.BlockSpec((bn, D), lambda i: (i, 0)),
        ],
        out_specs=pl.BlockSpec((bn, D), lambda i: (i, 0)),
        out_shape=jax.ShapeDtypeStruct((N, D), _f32),
    )(parts, root)


def kernel(x_src, x_target, edge_index, edge_type, target_node_type,
           src_node_type, W_rel, W_root, b_root):
    src = edge_index[0].astype(_i32)
    dst = edge_index[1].astype(_i32)
    typ = edge_type.astype(_i32)
    tnt2d = target_node_type.astype(_i32).reshape(N, 1)

    z, root = _tc_precompute(x_src.astype(_f32), x_target.astype(_f32),
                             W_rel.astype(_f32), W_root.astype(_f32),
                             b_root.astype(_f32), tnt2d)
    z = z.reshape(KEYS, D)
    gkey, ckey = _tc_keys(src, dst, typ)

    cnt = _sc_count(ckey).reshape(NC, KEYS_PAD)
    inv = _tc_inv(cnt).reshape(KEYS_PAD)

    parts = _sc_main(gkey, ckey, dst, z, inv)
    return _tc_final(parts, root)
